# baseline (speedup 1.0000x reference)
import jax
import jax.numpy as jnp
from jax import lax
from jax.experimental import pallas as pl
from jax.experimental.pallas import tpu as pltpu
from jax.experimental.pallas import tpu_sc as plsc

TOPK = 5
KTHR = 26214
LANES = 16
NEG = float("-inf")


def _image_kernel(x_ref, pv_ref, pi_ref, pc_ref, pa_ref):
    x = x_ref[0]
    H, W = x.shape
    col = lax.broadcasted_iota(jnp.int32, (H, W), 1)
    row = lax.broadcasted_iota(jnp.int32, (H, W), 0)

    def shift_cols(a, d):
        r = jnp.roll(a, -d, axis=1)
        valid = (col + d >= 0) & (col + d < W)
        return jnp.where(valid, r, NEG)

    def shift_rows(a, d):
        r = jnp.roll(a, -d, axis=0)
        valid = (row + d >= 0) & (row + d < H)
        return jnp.where(valid, r, NEG)

    m3 = jnp.maximum(jnp.maximum(shift_cols(x, -1), x), shift_cols(x, 1))
    m9 = jnp.maximum(jnp.maximum(shift_cols(m3, -3), m3), shift_cols(m3, 3))
    v3 = jnp.maximum(jnp.maximum(shift_rows(m9, -1), m9), shift_rows(m9, 1))
    lm = jnp.maximum(jnp.maximum(shift_rows(v3, -3), v3), shift_rows(v3, 3))

    peak = x == lm
    flat_idx = row * W + col
    masked = jnp.where(peak, x, NEG)

    big = jnp.int32(2**30)
    vals, idxs = [], []
    cur = masked
    for _ in range(TOPK):
        v = jnp.max(cur)
        idx = jnp.min(jnp.where(cur == v, flat_idx, big))
        vals.append(v)
        idxs.append(idx)
        cur = jnp.where(flat_idx == idx, NEG, cur)

    counts = [jnp.sum((x >= v).astype(jnp.int32)) for v in vals]

    pad = LANES - TOPK
    pv_ref[0, 0] = jnp.stack(vals + [jnp.float32(NEG)] * pad)
    pi_ref[0, 0] = jnp.stack(idxs + [jnp.int32(0)] * pad)
    pc_ref[0, 0] = jnp.stack(counts + [big] * pad)
    pa_ref[0, 0] = jnp.broadcast_to(0.5 * vals[0], (LANES,))


def _select_kernel(pv_hbm, pi_hbm, pc_hbm, pa_hbm, xo_hbm, yo_hbm, lo_hbm,
                   pv_v, pi_v, pc_v, pa_v, xo_v, yo_v, lo_v):
    c = lax.axis_index("c")
    s = lax.axis_index("s")

    @pl.when(c == 0)
    def _():
        pltpu.sync_copy(pv_hbm.at[s], pv_v)
        pltpu.sync_copy(pi_hbm.at[s], pi_v)
        pltpu.sync_copy(pc_hbm.at[s], pc_v)
        pltpu.sync_copy(pa_hbm.at[s], pa_v)
        vals = pv_v[...]
        idxs = pi_v[...]
        counts = pc_v[...]
        adapt = pa_v[...]
        lane = lax.iota(jnp.int32, LANES)
        valid = (counts < KTHR) & (vals >= adapt) & (lane < TOPK)
        keep = valid | (lane == 0)
        fx = jnp.bitwise_and(idxs, 511).astype(jnp.float32)
        fy = lax.shift_right_logical(idxs, 9).astype(jnp.float32)
        xo_v[...] = jnp.where(keep, fx, -1.0)
        yo_v[...] = jnp.where(keep, fy, -1.0)
        lo_v[...] = jnp.where(keep, 1.0, -1.0)
        pltpu.sync_copy(xo_v, xo_hbm.at[s])
        pltpu.sync_copy(yo_v, yo_hbm.at[s])
        pltpu.sync_copy(lo_v, lo_hbm.at[s])


def kernel(contrast_map):
    B, C, H, W = contrast_map.shape
    x = contrast_map.reshape(B, H, W)
    pv, pi, pc, pa = pl.pallas_call(
        _image_kernel,
        grid=(B,),
        in_specs=[pl.BlockSpec((1, H, W), lambda i: (i, 0, 0))],
        out_specs=[
            pl.BlockSpec((1, 1, LANES), lambda i: (i, 0, 0)),
            pl.BlockSpec((1, 1, LANES), lambda i: (i, 0, 0)),
            pl.BlockSpec((1, 1, LANES), lambda i: (i, 0, 0)),
            pl.BlockSpec((1, 1, LANES), lambda i: (i, 0, 0)),
        ],
        out_shape=[
            jax.ShapeDtypeStruct((B, 1, LANES), jnp.float32),
            jax.ShapeDtypeStruct((B, 1, LANES), jnp.int32),
            jax.ShapeDtypeStruct((B, 1, LANES), jnp.int32),
            jax.ShapeDtypeStruct((B, 1, LANES), jnp.float32),
        ],
    )(x)

    mesh = plsc.VectorSubcoreMesh(
        core_axis_name="c", subcore_axis_name="s", num_cores=2,
        num_subcores=16)
    fsel = pl.kernel(
        _select_kernel,
        out_type=[
            jax.ShapeDtypeStruct((B, LANES), jnp.float32),
            jax.ShapeDtypeStruct((B, LANES), jnp.float32),
            jax.ShapeDtypeStruct((B, LANES), jnp.float32),
        ],
        mesh=mesh,
        scratch_types=[
            pltpu.VMEM((LANES,), jnp.float32),
            pltpu.VMEM((LANES,), jnp.int32),
            pltpu.VMEM((LANES,), jnp.int32),
            pltpu.VMEM((LANES,), jnp.float32),
            pltpu.VMEM((LANES,), jnp.float32),
            pltpu.VMEM((LANES,), jnp.float32),
            pltpu.VMEM((LANES,), jnp.float32),
        ],
    )
    xs, ys, ls = fsel(pv.reshape(B, LANES), pi.reshape(B, LANES),
                      pc.reshape(B, LANES), pa.reshape(B, LANES))
    coords = jnp.stack([xs[:, :TOPK], ys[:, :TOPK]], axis=-1)
    labels = ls[:, :TOPK]
    return coords, labels


# speedup vs baseline: 1.3343x; 1.3343x over previous
import jax
import jax.numpy as jnp
from jax import lax
from jax.experimental import pallas as pl
from jax.experimental.pallas import tpu as pltpu
from jax.experimental.pallas import tpu_sc as plsc

TOPK = 5
KTHR = 26214
LANES = 16
IMGS_PER_STEP = 4
NEG = float("-inf")


def _image_kernel(x_ref, pv_ref, pi_ref, pc_ref, pa_ref):
    x = x_ref[...]
    T, H, W = x.shape
    col = lax.broadcasted_iota(jnp.int32, (T, H, W), 2)
    row = lax.broadcasted_iota(jnp.int32, (T, H, W), 1)

    def shift_cols(a, d):
        r = jnp.roll(a, -d, axis=2)
        valid = (col + d >= 0) & (col + d < W)
        return jnp.where(valid, r, NEG)

    def shift_rows(a, d):
        r = jnp.roll(a, -d, axis=1)
        valid = (row + d >= 0) & (row + d < H)
        return jnp.where(valid, r, NEG)

    m3 = jnp.maximum(jnp.maximum(shift_cols(x, -1), x), shift_cols(x, 1))
    m9 = jnp.maximum(jnp.maximum(shift_cols(m3, -3), m3), shift_cols(m3, 3))
    v3 = jnp.maximum(jnp.maximum(shift_rows(m9, -1), m9), shift_rows(m9, 1))
    lm = jnp.maximum(jnp.maximum(shift_rows(v3, -3), v3), shift_rows(v3, 3))

    peak = x == lm
    flat_idx = row * W + col
    masked = jnp.where(peak, x, NEG)

    big = jnp.int32(2**30)
    vals, idxs = [], []
    cur = masked
    for _ in range(TOPK):
        v = jnp.max(cur, axis=(1, 2))
        eq = cur == v[:, None, None]
        idx = jnp.min(jnp.where(eq, flat_idx, big), axis=(1, 2))
        vals.append(v)
        idxs.append(idx)
        cur = jnp.where(flat_idx == idx[:, None, None], NEG, cur)

    counts = [jnp.sum((x >= v[:, None, None]).astype(jnp.int32), axis=(1, 2))
              for v in vals]

    pad = LANES - TOPK
    pv = jnp.concatenate(
        [jnp.stack(vals, axis=1), jnp.full((T, pad), NEG, jnp.float32)], 1)
    pi = jnp.concatenate(
        [jnp.stack(idxs, axis=1), jnp.zeros((T, pad), jnp.int32)], 1)
    pc = jnp.concatenate(
        [jnp.stack(counts, axis=1), jnp.full((T, pad), big, jnp.int32)], 1)
    pv_ref[:, 0] = pv
    pi_ref[:, 0] = pi
    pc_ref[:, 0] = pc
    pa_ref[:, 0] = jnp.broadcast_to(0.5 * vals[0][:, None], (T, LANES))


def _select_kernel(pv_hbm, pi_hbm, pc_hbm, pa_hbm, xo_hbm, yo_hbm, lo_hbm,
                   pv_v, pi_v, pc_v, pa_v, xo_v, yo_v, lo_v):
    c = lax.axis_index("c")
    s = lax.axis_index("s")

    @pl.when(c == 0)
    def _():
        pltpu.sync_copy(pv_hbm.at[s], pv_v)
        pltpu.sync_copy(pi_hbm.at[s], pi_v)
        pltpu.sync_copy(pc_hbm.at[s], pc_v)
        pltpu.sync_copy(pa_hbm.at[s], pa_v)
        vals = pv_v[...]
        idxs = pi_v[...]
        counts = pc_v[...]
        adapt = pa_v[...]
        lane = lax.iota(jnp.int32, LANES)
        valid = (counts < KTHR) & (vals >= adapt) & (lane < TOPK)
        keep = valid | (lane == 0)
        fx = jnp.bitwise_and(idxs, 511).astype(jnp.float32)
        fy = lax.shift_right_logical(idxs, 9).astype(jnp.float32)
        xo_v[...] = jnp.where(keep, fx, -1.0)
        yo_v[...] = jnp.where(keep, fy, -1.0)
        lo_v[...] = jnp.where(keep, 1.0, -1.0)
        pltpu.sync_copy(xo_v, xo_hbm.at[s])
        pltpu.sync_copy(yo_v, yo_hbm.at[s])
        pltpu.sync_copy(lo_v, lo_hbm.at[s])


def kernel(contrast_map):
    B, C, H, W = contrast_map.shape
    x = contrast_map.reshape(B, H, W)
    t = IMGS_PER_STEP
    pv, pi, pc, pa = pl.pallas_call(
        _image_kernel,
        grid=(B // t,),
        in_specs=[pl.BlockSpec((t, H, W), lambda i: (i, 0, 0))],
        out_specs=[
            pl.BlockSpec((t, 1, LANES), lambda i: (i, 0, 0)),
            pl.BlockSpec((t, 1, LANES), lambda i: (i, 0, 0)),
            pl.BlockSpec((t, 1, LANES), lambda i: (i, 0, 0)),
            pl.BlockSpec((t, 1, LANES), lambda i: (i, 0, 0)),
        ],
        out_shape=[
            jax.ShapeDtypeStruct((B, 1, LANES), jnp.float32),
            jax.ShapeDtypeStruct((B, 1, LANES), jnp.int32),
            jax.ShapeDtypeStruct((B, 1, LANES), jnp.int32),
            jax.ShapeDtypeStruct((B, 1, LANES), jnp.float32),
        ],
    )(x)

    mesh = plsc.VectorSubcoreMesh(
        core_axis_name="c", subcore_axis_name="s", num_cores=2,
        num_subcores=16)
    fsel = pl.kernel(
        _select_kernel,
        out_type=[
            jax.ShapeDtypeStruct((B, LANES), jnp.float32),
            jax.ShapeDtypeStruct((B, LANES), jnp.float32),
            jax.ShapeDtypeStruct((B, LANES), jnp.float32),
        ],
        mesh=mesh,
        scratch_types=[
            pltpu.VMEM((LANES,), jnp.float32),
            pltpu.VMEM((LANES,), jnp.int32),
            pltpu.VMEM((LANES,), jnp.int32),
            pltpu.VMEM((LANES,), jnp.float32),
            pltpu.VMEM((LANES,), jnp.float32),
            pltpu.VMEM((LANES,), jnp.float32),
            pltpu.VMEM((LANES,), jnp.float32),
        ],
    )
    xs, ys, ls = fsel(pv.reshape(B, LANES), pi.reshape(B, LANES),
                      pc.reshape(B, LANES), pa.reshape(B, LANES))
    coords = jnp.stack([xs[:, :TOPK], ys[:, :TOPK]], axis=-1)
    labels = ls[:, :TOPK]
    return coords, labels


# speedup vs baseline: 1.3839x; 1.0371x over previous
import jax
import jax.numpy as jnp
from jax import lax
from jax.experimental import pallas as pl
from jax.experimental.pallas import tpu as pltpu
from jax.experimental.pallas import tpu_sc as plsc

TOPK = 5
KTHR = 26214
LANES = 16
IMGS_PER_STEP = 4
NEG = float("-inf")


def _image_kernel(x_ref, pk_ref):
    x = x_ref[...]
    T, H, W = x.shape
    col = lax.broadcasted_iota(jnp.int32, (T, H, W), 2)
    row = lax.broadcasted_iota(jnp.int32, (T, H, W), 1)

    def shift_cols(a, d):
        r = jnp.roll(a, -d, axis=2)
        valid = (col + d >= 0) & (col + d < W)
        return jnp.where(valid, r, NEG)

    def shift_rows(a, d):
        r = jnp.roll(a, -d, axis=1)
        valid = (row + d >= 0) & (row + d < H)
        return jnp.where(valid, r, NEG)

    m3 = jnp.maximum(jnp.maximum(shift_cols(x, -1), x), shift_cols(x, 1))
    m9 = jnp.maximum(jnp.maximum(shift_cols(m3, -3), m3), shift_cols(m3, 3))
    v3 = jnp.maximum(jnp.maximum(shift_rows(m9, -1), m9), shift_rows(m9, 1))
    lm = jnp.maximum(jnp.maximum(shift_rows(v3, -3), v3), shift_rows(v3, 3))

    peak = x == lm
    flat_idx = row * W + col
    masked = jnp.where(peak, x, NEG)

    big = jnp.int32(2**30)
    vals, idxs = [], []
    cur = masked
    for _ in range(TOPK):
        v = jnp.max(cur, axis=(1, 2))
        eq = cur == v[:, None, None]
        idx = jnp.min(jnp.where(eq, flat_idx, big), axis=(1, 2))
        vals.append(v)
        idxs.append(idx)
        cur = jnp.where(flat_idx == idx[:, None, None], NEG, cur)

    counts = [jnp.sum((x >= v[:, None, None]).astype(jnp.int32), axis=(1, 2))
              for v in vals]

    pad = LANES - TOPK
    pv = jnp.concatenate(
        [jnp.stack(vals, axis=1), jnp.full((T, pad), NEG, jnp.float32)], 1)
    pi = jnp.concatenate(
        [jnp.stack(idxs, axis=1), jnp.zeros((T, pad), jnp.int32)], 1)
    pc = jnp.concatenate(
        [jnp.stack(counts, axis=1), jnp.full((T, pad), big, jnp.int32)], 1)
    pa = jnp.broadcast_to(0.5 * vals[0][:, None], (T, LANES))
    packed = jnp.concatenate(
        [pv, pi.astype(jnp.float32), pc.astype(jnp.float32), pa], axis=1)
    pk_ref[:, 0] = packed


def _select_kernel(pk_hbm, out_hbm, pk_v, out_v):
    c = lax.axis_index("c")
    s = lax.axis_index("s")

    @pl.when(c == 0)
    def _():
        pltpu.sync_copy(pk_hbm.at[s], pk_v)
        vals = pk_v[pl.ds(0, LANES)]
        idxs = pk_v[pl.ds(LANES, LANES)].astype(jnp.int32)
        counts = pk_v[pl.ds(2 * LANES, LANES)].astype(jnp.int32)
        adapt = pk_v[pl.ds(3 * LANES, LANES)]
        lane = lax.iota(jnp.int32, LANES)
        valid = (counts < KTHR) & (vals >= adapt) & (lane < TOPK)
        keep = valid | (lane == 0)
        fx = jnp.bitwise_and(idxs, 511).astype(jnp.float32)
        fy = lax.shift_right_logical(idxs, 9).astype(jnp.float32)
        out_v[pl.ds(0, LANES)] = jnp.where(keep, fx, -1.0)
        out_v[pl.ds(LANES, LANES)] = jnp.where(keep, fy, -1.0)
        out_v[pl.ds(2 * LANES, LANES)] = jnp.where(keep, 1.0, -1.0)
        pltpu.sync_copy(out_v, out_hbm.at[s])


def kernel(contrast_map):
    B, C, H, W = contrast_map.shape
    x = contrast_map.reshape(B, H, W)
    t = IMGS_PER_STEP
    pk = pl.pallas_call(
        _image_kernel,
        grid=(B // t,),
        in_specs=[pl.BlockSpec((t, H, W), lambda i: (i, 0, 0))],
        out_specs=[pl.BlockSpec((t, 1, 4 * LANES), lambda i: (i, 0, 0))],
        out_shape=[jax.ShapeDtypeStruct((B, 1, 4 * LANES), jnp.float32)],
    )(x)[0]

    mesh = plsc.VectorSubcoreMesh(
        core_axis_name="c", subcore_axis_name="s", num_cores=2,
        num_subcores=16)
    fsel = pl.kernel(
        _select_kernel,
        out_type=[jax.ShapeDtypeStruct((B, 3 * LANES), jnp.float32)],
        mesh=mesh,
        scratch_types=[
            pltpu.VMEM((4 * LANES,), jnp.float32),
            pltpu.VMEM((3 * LANES,), jnp.float32),
        ],
    )
    out = fsel(pk.reshape(B, 4 * LANES))[0]
    coords = jnp.stack(
        [out[:, 0:TOPK], out[:, LANES:LANES + TOPK]], axis=-1)
    labels = out[:, 2 * LANES:2 * LANES + TOPK]
    return coords, labels


# speedup vs baseline: 1.4034x; 1.0141x over previous
import jax
import jax.numpy as jnp
from jax import lax
from jax.experimental import pallas as pl
from jax.experimental.pallas import tpu as pltpu
from jax.experimental.pallas import tpu_sc as plsc

TOPK = 5
KTHR = 26214
LANES = 16
IMGS_PER_STEP = 4
NEG = float("-inf")


def _image_kernel(x_ref, pk_ref):
    x = x_ref[...]
    T, H, W = x.shape
    col = lax.broadcasted_iota(jnp.int32, (T, H, W), 2)
    row = lax.broadcasted_iota(jnp.int32, (T, H, W), 1)

    def shift_cols(a, d):
        r = jnp.roll(a, -d, axis=2)
        valid = (col + d >= 0) & (col + d < W)
        return jnp.where(valid, r, NEG)

    def shift_rows(a, d):
        r = jnp.roll(a, -d, axis=1)
        valid = (row + d >= 0) & (row + d < H)
        return jnp.where(valid, r, NEG)

    m3 = jnp.maximum(jnp.maximum(shift_cols(x, -1), x), shift_cols(x, 1))
    m9 = jnp.maximum(jnp.maximum(shift_cols(m3, -3), m3), shift_cols(m3, 3))
    v3 = jnp.maximum(jnp.maximum(shift_rows(m9, -1), m9), shift_rows(m9, 1))
    lm = jnp.maximum(jnp.maximum(shift_rows(v3, -3), v3), shift_rows(v3, 3))

    peak = x == lm
    flat_idx = row * W + col
    masked = jnp.where(peak, x, NEG)

    big = jnp.int32(2**30)
    vals, idxs = [], []
    cur = masked
    for _ in range(TOPK):
        v = jnp.max(cur, axis=(1, 2))
        eq = cur == v[:, None, None]
        idx = jnp.min(jnp.where(eq, flat_idx, big), axis=(1, 2))
        vals.append(v)
        idxs.append(idx)
        cur = jnp.where(flat_idx == idx[:, None, None], NEG, cur)

    counts = [jnp.zeros((T,), jnp.int32)] + [
        jnp.sum((x >= v[:, None, None]).astype(jnp.int32), axis=(1, 2))
        for v in vals[1:]]

    pad = LANES - TOPK
    pv = jnp.concatenate(
        [jnp.stack(vals, axis=1), jnp.full((T, pad), NEG, jnp.float32)], 1)
    pi = jnp.concatenate(
        [jnp.stack(idxs, axis=1), jnp.zeros((T, pad), jnp.int32)], 1)
    pc = jnp.concatenate(
        [jnp.stack(counts, axis=1), jnp.full((T, pad), big, jnp.int32)], 1)
    pa = jnp.broadcast_to(0.5 * vals[0][:, None], (T, LANES))
    packed = jnp.concatenate(
        [pv, pi.astype(jnp.float32), pc.astype(jnp.float32), pa], axis=1)
    pk_ref[:, 0] = packed


def _select_kernel(pk_hbm, out_hbm, pk_v, out_v):
    c = lax.axis_index("c")
    s = lax.axis_index("s")

    @pl.when(c == 0)
    def _():
        pltpu.sync_copy(pk_hbm.at[s], pk_v)
        vals = pk_v[pl.ds(0, LANES)]
        idxs = pk_v[pl.ds(LANES, LANES)].astype(jnp.int32)
        counts = pk_v[pl.ds(2 * LANES, LANES)].astype(jnp.int32)
        adapt = pk_v[pl.ds(3 * LANES, LANES)]
        lane = lax.iota(jnp.int32, LANES)
        valid = (counts < KTHR) & (vals >= adapt) & (lane < TOPK)
        keep = valid | (lane == 0)
        fx = jnp.bitwise_and(idxs, 511).astype(jnp.float32)
        fy = lax.shift_right_logical(idxs, 9).astype(jnp.float32)
        out_v[pl.ds(0, LANES)] = jnp.where(keep, fx, -1.0)
        out_v[pl.ds(LANES, LANES)] = jnp.where(keep, fy, -1.0)
        out_v[pl.ds(2 * LANES, LANES)] = jnp.where(keep, 1.0, -1.0)
        pltpu.sync_copy(out_v, out_hbm.at[s])


def kernel(contrast_map):
    B, C, H, W = contrast_map.shape
    x = contrast_map.reshape(B, H, W)
    t = IMGS_PER_STEP
    pk = pl.pallas_call(
        _image_kernel,
        grid=(B // t,),
        in_specs=[pl.BlockSpec((t, H, W), lambda i: (i, 0, 0))],
        out_specs=[pl.BlockSpec((t, 1, 4 * LANES), lambda i: (i, 0, 0))],
        out_shape=[jax.ShapeDtypeStruct((B, 1, 4 * LANES), jnp.float32)],
    )(x)[0]

    mesh = plsc.VectorSubcoreMesh(
        core_axis_name="c", subcore_axis_name="s", num_cores=2,
        num_subcores=16)
    fsel = pl.kernel(
        _select_kernel,
        out_type=[jax.ShapeDtypeStruct((B, 3 * LANES), jnp.float32)],
        mesh=mesh,
        scratch_types=[
            pltpu.VMEM((4 * LANES,), jnp.float32),
            pltpu.VMEM((3 * LANES,), jnp.float32),
        ],
    )
    out = fsel(pk.reshape(B, 4 * LANES))[0]
    coords = jnp.stack(
        [out[:, 0:TOPK], out[:, LANES:LANES + TOPK]], axis=-1)
    labels = out[:, 2 * LANES:2 * LANES + TOPK]
    return coords, labels
